# baseline (speedup 1.0000x reference)
import functools

import jax
import jax.numpy as jnp
from jax import lax
from jax.experimental import pallas as pl
from jax.experimental.pallas import tpu as pltpu
from jax.experimental.pallas import tpu_sc as plsc

NC = 2
NS = 16
NW = NC * NS
L = 16


def _rsqrt(x):
    i = plsc.bitcast(x, jnp.int32)
    i = jnp.int32(0x5F3759DF) - lax.shift_right_arithmetic(i, 1)
    y = plsc.bitcast(i, jnp.float32)
    for _ in range(3):
        y = y * (jnp.float32(1.5) - jnp.float32(0.5) * x * y * y)
    return y


def _sqrt(x):
    return x * _rsqrt(x)


def _log(z):
    zi = plsc.bitcast(z, jnp.int32)
    ex = lax.shift_right_arithmetic(zi, 23) - jnp.int32(127)
    mi = (zi & jnp.int32(0x007FFFFF)) | jnp.int32(0x3F800000)
    m = plsc.bitcast(mi, jnp.float32)
    big = m > jnp.float32(1.4142135)
    m = jnp.where(big, m * jnp.float32(0.5), m)
    ex = ex + jnp.where(big, jnp.int32(1), jnp.int32(0))
    s = (m - jnp.float32(1.0)) / (m + jnp.float32(1.0))
    s2 = s * s
    p = s2 * jnp.float32(1.0 / 9.0)
    for c in (1.0 / 7.0, 1.0 / 5.0, 1.0 / 3.0, 1.0):
        p = s2 * p + jnp.float32(c)
    p = jnp.float32(2.0) * s * p
    return ex.astype(jnp.float32) * jnp.float32(0.6931471805599453) + p


def _make_sc_kernel(N, D, B):
    assert D == 64 and B % NW == 0
    bpw = B // NW
    ngrp = bpw // L
    nch = bpw // 128
    dh = D // 2
    mesh = plsc.VectorSubcoreMesh(core_axis_name="c", subcore_axis_name="s",
                                  num_cores=NC, num_subcores=NS)

    @functools.partial(
        pl.kernel,
        mesh=mesh,
        out_type=jax.ShapeDtypeStruct((B,), jnp.float32),
        compiler_params=pltpu.CompilerParams(needs_layout_passes=False,
                                             use_tc_tiling_on_sc=False),
        scratch_types=[
            pltpu.VMEM((dh, L), jnp.float32),
            pltpu.VMEM((dh, L), jnp.float32),
            pltpu.VMEM((bpw,), jnp.int32),
            pltpu.VMEM((bpw,), jnp.int32),
            pltpu.VMEM((D, bpw), jnp.float32),
            pltpu.VMEM((D, bpw), jnp.float32),
            pltpu.VMEM((bpw,), jnp.float32),
            pltpu.VMEM((bpw,), jnp.float32),
            pltpu.VMEM((bpw,), jnp.float32),
            pltpu.VMEM((bpw,), jnp.float32),
            pltpu.SemaphoreType.DMA,
        ],
    )
    def sc_kernel(cb_hbm, sb_hbm, uidx_hbm, vidx_hbm, w_hbm, euclT_hbm,
                  bias_hbm, out_hbm, cb_v, sb_v, uidx_v, vidx_v, gu, gv,
                  w_v, bu_v, bv_v, out_v, sem):
        wid = lax.axis_index("s") * NC + lax.axis_index("c")
        base = wid * bpw
        pltpu.sync_copy(cb_hbm, cb_v)
        pltpu.sync_copy(sb_hbm, sb_v)
        pltpu.sync_copy(uidx_hbm.at[pl.ds(base, bpw)], uidx_v)
        pltpu.sync_copy(vidx_hbm.at[pl.ds(base, bpw)], vidx_v)
        pltpu.sync_copy(w_hbm.at[pl.ds(base, bpw)], w_v)

        cps = []
        for k in range(nch):
            sl = pl.ds(k * 128, 128)
            cps.append(pltpu.async_copy(
                bias_hbm.at[uidx_v.at[sl]], bu_v.at[sl], sem))
            cps.append(pltpu.async_copy(
                bias_hbm.at[vidx_v.at[sl]], bv_v.at[sl], sem))

        def feat(j, carry):
            fcps = []
            for k in range(nch):
                sl = pl.ds(k * 128, 128)
                fcps.append(pltpu.async_copy(
                    euclT_hbm.at[j].at[uidx_v.at[sl]], gu.at[j].at[sl], sem))
                fcps.append(pltpu.async_copy(
                    euclT_hbm.at[j].at[vidx_v.at[sl]], gv.at[j].at[sl], sem))
            for cp in fcps:
                cp.wait()
            return carry

        lax.fori_loop(0, D, feat, 0)
        for cp in cps:
            cp.wait()

        def group(g, carry):
            p0 = g * L
            psl = pl.ds(p0, L)
            nu = jnp.zeros((L,), jnp.float32)
            nv = jnp.zeros((L,), jnp.float32)
            dot = jnp.zeros((L,), jnp.float32)
            for j in range(dh):
                ue = gu[2 * j, psl]
                uo = gu[2 * j + 1, psl]
                ve = gv[2 * j, psl]
                vo = gv[2 * j + 1, psl]
                cj = cb_v[j, :]
                sj = sb_v[j, :]
                nu = nu + (ue * ue + uo * uo)
                nv = nv + (ve * ve + vo * vo)
                dot = dot + cj * (ue * ve + uo * vo) + sj * (uo * ve - ue * vo)
            x0u = _sqrt(jnp.float32(1.0) + nu)
            x0v = _sqrt(jnp.float32(1.0) + nv)
            minner = x0u * x0v - dot
            arg = jnp.maximum(minner, jnp.float32(1.0 + 1e-7))
            e = arg - jnp.float32(1.0)
            t = e + _sqrt(e * (e + jnp.float32(2.0)))
            d = _log(jnp.float32(1.0) + t)
            wv = w_v[psl]
            out_v[psl] = -wv * d * d + bu_v[psl] + bv_v[psl]
            return carry

        lax.fori_loop(0, ngrp, group, 0)
        pltpu.sync_copy(out_v, out_hbm.at[pl.ds(base, bpw)])

    return sc_kernel


def kernel(u_idx, v_idx, w_uv, theta_src, theta_dst, eucl, bias):
    N, D = eucl.shape
    B = u_idx.shape[0]
    phi = theta_dst - theta_src
    cb = jnp.broadcast_to(jnp.cos(phi)[:, None], (D // 2, L))
    sb = jnp.broadcast_to(jnp.sin(phi)[:, None], (D // 2, L))
    sc = _make_sc_kernel(N, D, B)
    return sc(cb.astype(jnp.float32), sb.astype(jnp.float32),
              u_idx.astype(jnp.int32), v_idx.astype(jnp.int32),
              w_uv, eucl.T, bias)


# speedup vs baseline: 8.7460x; 8.7460x over previous
import functools

import jax
import jax.numpy as jnp
from jax import lax
from jax.experimental import pallas as pl
from jax.experimental.pallas import tpu as pltpu
from jax.experimental.pallas import tpu_sc as plsc

NC = 2
NS = 16
NW = NC * NS
L = 16


def _rsqrt(x):
    i = plsc.bitcast(x, jnp.int32)
    i = jnp.int32(0x5F3759DF) - lax.shift_right_arithmetic(i, 1)
    y = plsc.bitcast(i, jnp.float32)
    for _ in range(3):
        y = y * (jnp.float32(1.5) - jnp.float32(0.5) * x * y * y)
    return y


def _sqrt(x):
    return x * _rsqrt(x)


def _log(z):
    zi = plsc.bitcast(z, jnp.int32)
    ex = lax.shift_right_arithmetic(zi, 23) - jnp.int32(127)
    mi = (zi & jnp.int32(0x007FFFFF)) | jnp.int32(0x3F800000)
    m = plsc.bitcast(mi, jnp.float32)
    big = m > jnp.float32(1.4142135)
    m = jnp.where(big, m * jnp.float32(0.5), m)
    ex = ex + jnp.where(big, jnp.int32(1), jnp.int32(0))
    s = (m - jnp.float32(1.0)) / (m + jnp.float32(1.0))
    s2 = s * s
    p = s2 * jnp.float32(1.0 / 9.0)
    for c in (1.0 / 7.0, 1.0 / 5.0, 1.0 / 3.0, 1.0):
        p = s2 * p + jnp.float32(c)
    p = jnp.float32(2.0) * s * p
    return ex.astype(jnp.float32) * jnp.float32(0.6931471805599453) + p


def _make_sc_kernel(N, D, B):
    assert D == 64 and B % NW == 0
    bpw = B // NW
    ngrp = bpw // L
    nch = bpw // 128
    dh = D // 2
    mesh = plsc.VectorSubcoreMesh(core_axis_name="c", subcore_axis_name="s",
                                  num_cores=NC, num_subcores=NS)

    @functools.partial(
        pl.kernel,
        mesh=mesh,
        out_type=jax.ShapeDtypeStruct((B,), jnp.float32),
        compiler_params=pltpu.CompilerParams(needs_layout_passes=False,
                                             use_tc_tiling_on_sc=False),
        scratch_types=[
            pltpu.VMEM((dh, L), jnp.float32),
            pltpu.VMEM((dh, L), jnp.float32),
            pltpu.VMEM((bpw,), jnp.int32),
            pltpu.VMEM((bpw,), jnp.int32),
            pltpu.VMEM((bpw,), jnp.int32),
            pltpu.VMEM((bpw,), jnp.int32),
            pltpu.VMEM((bpw, D), jnp.float32),
            pltpu.VMEM((bpw, D), jnp.float32),
            pltpu.VMEM((bpw,), jnp.float32),
            pltpu.VMEM((bpw,), jnp.float32),
            pltpu.VMEM((bpw,), jnp.float32),
            pltpu.VMEM((bpw,), jnp.float32),
            pltpu.SemaphoreType.DMA,
        ],
    )
    def sc_kernel(cb_hbm, sb_hbm, uidx_hbm, vidx_hbm, uidx2_hbm, vidx2_hbm,
                  w_hbm, eucl_hbm, bias_hbm, out_hbm, cb_v, sb_v, uidx_v,
                  vidx_v, uidx2_v, vidx2_v, rows_u, rows_v, w_v, bu_v, bv_v,
                  out_v, sem):
        wid = lax.axis_index("s") * NC + lax.axis_index("c")
        base = wid * bpw
        pltpu.sync_copy(cb_hbm, cb_v)
        pltpu.sync_copy(sb_hbm, sb_v)
        pltpu.sync_copy(uidx_hbm.at[pl.ds(base, bpw)], uidx_v)
        pltpu.sync_copy(vidx_hbm.at[pl.ds(base, bpw)], vidx_v)
        pltpu.sync_copy(uidx2_hbm.at[pl.ds(base, bpw)], uidx2_v)
        pltpu.sync_copy(vidx2_hbm.at[pl.ds(base, bpw)], vidx2_v)
        pltpu.sync_copy(w_hbm.at[pl.ds(base, bpw)], w_v)
        cps = []
        for k in range(nch):
            sl = pl.ds(k * 128, 128)
            cps.append(pltpu.async_copy(
                eucl_hbm.at[uidx2_v.at[sl]], rows_u.at[sl], sem))
            cps.append(pltpu.async_copy(
                eucl_hbm.at[vidx2_v.at[sl]], rows_v.at[sl], sem))
            cps.append(pltpu.async_copy(
                bias_hbm.at[uidx_v.at[sl]], bu_v.at[sl], sem))
            cps.append(pltpu.async_copy(
                bias_hbm.at[vidx_v.at[sl]], bv_v.at[sl], sem))
        for cp in cps:
            cp.wait()

        def group(g, carry):
            p0 = g * L
            idx_p = p0 + lax.iota(jnp.int32, L)
            nu = jnp.zeros((L,), jnp.float32)
            nv = jnp.zeros((L,), jnp.float32)
            dot = jnp.zeros((L,), jnp.float32)
            for j in range(dh):
                de = jnp.full((L,), 2 * j, jnp.int32)
                do = jnp.full((L,), 2 * j + 1, jnp.int32)
                ue = plsc.load_gather(rows_u, [idx_p, de])
                uo = plsc.load_gather(rows_u, [idx_p, do])
                ve = plsc.load_gather(rows_v, [idx_p, de])
                vo = plsc.load_gather(rows_v, [idx_p, do])
                cj = cb_v[j, :]
                sj = sb_v[j, :]
                nu = nu + (ue * ue + uo * uo)
                nv = nv + (ve * ve + vo * vo)
                dot = dot + cj * (ue * ve + uo * vo) + sj * (uo * ve - ue * vo)
            x0u = _sqrt(jnp.float32(1.0) + nu)
            x0v = _sqrt(jnp.float32(1.0) + nv)
            minner = x0u * x0v - dot
            arg = jnp.maximum(minner, jnp.float32(1.0 + 1e-7))
            e = arg - jnp.float32(1.0)
            t = e + _sqrt(e * (e + jnp.float32(2.0)))
            d = _log(jnp.float32(1.0) + t)
            wv = w_v[pl.ds(p0, L)]
            out_v[pl.ds(p0, L)] = (-wv * d * d + bu_v[pl.ds(p0, L)]
                                   + bv_v[pl.ds(p0, L)])
            return carry

        lax.fori_loop(0, ngrp, group, 0)
        pltpu.sync_copy(out_v, out_hbm.at[pl.ds(base, bpw)])

    return sc_kernel


def kernel(u_idx, v_idx, w_uv, theta_src, theta_dst, eucl, bias):
    N, D = eucl.shape
    B = u_idx.shape[0]
    phi = theta_dst - theta_src
    cb = jnp.broadcast_to(jnp.cos(phi)[:, None], (D // 2, L))
    sb = jnp.broadcast_to(jnp.sin(phi)[:, None], (D // 2, L))
    eucl2 = jnp.pad(eucl, ((0, 0), (0, 128 - D))).reshape(2 * N, D)
    ui = u_idx.astype(jnp.int32)
    vi = v_idx.astype(jnp.int32)
    sc = _make_sc_kernel(N, D, B)
    return sc(cb.astype(jnp.float32), sb.astype(jnp.float32),
              ui, vi, ui * 2, vi * 2, w_uv, eucl2, bias)
